# baseline (speedup 1.0000x reference)
import functools

import jax
import jax.numpy as jnp
from jax import lax
from jax.experimental import pallas as pl
from jax.experimental.pallas import tpu as pltpu
from jax.experimental.pallas import tpu_sc as plsc

D = 32
R = 4
DW = 128
L = 16
NW = 32
CH = 160


def _dot_sigmoid_chunk(idxbuf, rows_a, rows_b, outbuf):
    def group(g, carry):
        row_ids = g * L + lax.iota(jnp.int32, L)
        lane = lax.iota(jnp.int32, L)
        idx16 = idxbuf[pl.ds(g * L, L)]
        colbase = (idx16 & (R - 1)) << 5
        accs = [jnp.zeros((L,), jnp.float32) for _ in range(4)]
        for d in range(D):
            col = colbase + ((lane + d) & (D - 1))
            a = plsc.load_gather(rows_a, [row_ids, col])
            b = plsc.load_gather(rows_b, [row_ids, col])
            accs[d % 4] = accs[d % 4] + a * b
        acc = (accs[0] + accs[1]) + (accs[2] + accs[3])
        outbuf[pl.ds(g * L, L)] = 1.0 / (1.0 + jnp.exp(-acc))
        return carry

    lax.fori_loop(0, CH // L, group, None)


def _shift_idx(src, dst):
    def body(i, carry):
        dst[pl.ds(i * L, L)] = src[pl.ds(i * L, L)] >> 2
        return carry

    lax.fori_loop(0, CH // L, body, None)


def _make_pass(n):
    assert n % (NW * 2 * CH) == 0
    mesh = plsc.VectorSubcoreMesh(core_axis_name="c", subcore_axis_name="s")

    @functools.partial(
        pl.kernel,
        out_type=jax.ShapeDtypeStruct((n,), jnp.float32),
        mesh=mesh,
        compiler_params=pltpu.CompilerParams(
            needs_layout_passes=False, use_tc_tiling_on_sc=True),
        scratch_types=[
            pltpu.VMEM((CH,), jnp.int32),
            pltpu.VMEM((CH,), jnp.int32),
            pltpu.VMEM((CH,), jnp.int32),
            pltpu.VMEM((CH,), jnp.int32),
            pltpu.VMEM((CH,), jnp.int32),
            pltpu.VMEM((CH,), jnp.int32),
            pltpu.VMEM((CH,), jnp.int32),
            pltpu.VMEM((CH,), jnp.int32),
            pltpu.VMEM((CH, DW), jnp.float32),
            pltpu.VMEM((CH, DW), jnp.float32),
            pltpu.VMEM((CH, DW), jnp.float32),
            pltpu.VMEM((CH, DW), jnp.float32),
            pltpu.VMEM((CH,), jnp.float32),
            pltpu.SemaphoreType.DMA,
            pltpu.SemaphoreType.DMA,
            pltpu.SemaphoreType.DMA,
            pltpu.SemaphoreType.DMA,
        ],
    )
    def run(tab_a_h, tab_b_h, ia_h, ib_h, out_h,
            ia0, ia1, ib0, ib1, qa0, qa1, qb0, qb1,
            ra0, ra1, rb0, rb1, outbuf,
            sa0, sa1, sb0, sb1):
        wid = lax.axis_index("s") * 2 + lax.axis_index("c")
        idx_a = (ia0, ia1)
        idx_b = (ib0, ib1)
        q_a = (qa0, qa1)
        q_b = (qb0, qb1)
        rows_a = (ra0, ra1)
        rows_b = (rb0, rb1)
        sem_a = (sa0, sa1)
        sem_b = (sb0, sb1)

        per_w = n // NW
        nch = per_w // CH
        base_w = wid * per_w

        def issue(c, k):
            base = base_w + c * CH
            pltpu.sync_copy(ia_h.at[pl.ds(base, CH)], idx_a[k])
            pltpu.sync_copy(ib_h.at[pl.ds(base, CH)], idx_b[k])
            _shift_idx(idx_a[k], q_a[k])
            _shift_idx(idx_b[k], q_b[k])
            pltpu.async_copy(tab_a_h.at[q_a[k]], rows_a[k], sem_a[k])
            pltpu.async_copy(tab_b_h.at[q_b[k]], rows_b[k], sem_b[k])

        def drain(k):
            pltpu.make_async_copy(
                tab_a_h.at[q_a[k]], rows_a[k], sem_a[k]).wait()
            pltpu.make_async_copy(
                tab_b_h.at[q_b[k]], rows_b[k], sem_b[k]).wait()

        def finish(c, k):
            drain(k)
            _dot_sigmoid_chunk(idx_a[k], rows_a[k], rows_b[k], outbuf)
            pltpu.sync_copy(outbuf, out_h.at[pl.ds(base_w + c * CH, CH)])

        issue(0, 0)

        def pair(p, carry):
            c0 = p * 2
            issue(c0 + 1, 1)
            finish(c0, 0)
            @pl.when(c0 + 2 < nch)
            def _():
                issue(c0 + 2, 0)
            finish(c0 + 1, 1)
            return carry

        lax.fori_loop(0, nch // 2, pair, None)

    return run


def kernel(B, V, W, user_idx, item_idx, user_idx2, trust_idx2):
    Bq = B.reshape(B.shape[0] // R, DW)
    Vq = V.reshape(V.shape[0] // R, DW)
    Wq = W.reshape(W.shape[0] // R, DW)
    pred_r = _make_pass(user_idx.shape[0])(Bq, Vq, user_idx, item_idx)
    pred_t = _make_pass(user_idx2.shape[0])(Bq, Wq, user_idx2, trust_idx2)
    return (pred_r, pred_t)


# speedup vs baseline: 1.0953x; 1.0953x over previous
import functools

import jax
import jax.numpy as jnp
from jax import lax
from jax.experimental import pallas as pl
from jax.experimental.pallas import tpu as pltpu
from jax.experimental.pallas import tpu_sc as plsc

D = 32
L = 16
NW = 32
CH = 640


def _dot_sigmoid_chunk(rows_a, rows_b, outbuf):
    def group(g, carry):
        row_ids = g * L + lax.iota(jnp.int32, L)
        lane = lax.iota(jnp.int32, L)
        accs = [jnp.zeros((L,), jnp.float32) for _ in range(4)]
        for d in range(D):
            col = (lane + d) & (D - 1)
            a = plsc.load_gather(rows_a, [row_ids, col])
            b = plsc.load_gather(rows_b, [row_ids, col])
            accs[d % 4] = accs[d % 4] + a * b
        acc = (accs[0] + accs[1]) + (accs[2] + accs[3])
        outbuf[pl.ds(g * L, L)] = 1.0 / (1.0 + jnp.exp(-acc))
        return carry

    lax.fori_loop(0, CH // L, group, None)


def _make_pass(n):
    assert n % (NW * 2 * CH) == 0
    mesh = plsc.VectorSubcoreMesh(core_axis_name="c", subcore_axis_name="s")

    @functools.partial(
        pl.kernel,
        out_type=jax.ShapeDtypeStruct((n,), jnp.float32),
        mesh=mesh,
        compiler_params=pltpu.CompilerParams(
            needs_layout_passes=False, use_tc_tiling_on_sc=False),
        scratch_types=[
            pltpu.VMEM((CH,), jnp.int32),
            pltpu.VMEM((CH,), jnp.int32),
            pltpu.VMEM((CH,), jnp.int32),
            pltpu.VMEM((CH,), jnp.int32),
            pltpu.VMEM((CH, D), jnp.float32),
            pltpu.VMEM((CH, D), jnp.float32),
            pltpu.VMEM((CH, D), jnp.float32),
            pltpu.VMEM((CH, D), jnp.float32),
            pltpu.VMEM((CH,), jnp.float32),
            pltpu.SemaphoreType.DMA,
            pltpu.SemaphoreType.DMA,
            pltpu.SemaphoreType.DMA,
            pltpu.SemaphoreType.DMA,
        ],
    )
    def run(tab_a_h, tab_b_h, ia_h, ib_h, out_h,
            ia0, ia1, ib0, ib1, ra0, ra1, rb0, rb1, outbuf,
            sa0, sa1, sb0, sb1):
        wid = lax.axis_index("s") * 2 + lax.axis_index("c")
        idx_a = (ia0, ia1)
        idx_b = (ib0, ib1)
        rows_a = (ra0, ra1)
        rows_b = (rb0, rb1)
        sem_a = (sa0, sa1)
        sem_b = (sb0, sb1)

        per_w = n // NW
        nch = per_w // CH
        base_w = wid * per_w

        def issue(c, k):
            base = base_w + c * CH
            pltpu.sync_copy(ia_h.at[pl.ds(base, CH)], idx_a[k])
            pltpu.sync_copy(ib_h.at[pl.ds(base, CH)], idx_b[k])
            pltpu.async_copy(tab_a_h.at[idx_a[k]], rows_a[k], sem_a[k])
            pltpu.async_copy(tab_b_h.at[idx_b[k]], rows_b[k], sem_b[k])

        def drain(k):
            pltpu.make_async_copy(
                tab_a_h.at[idx_a[k]], rows_a[k], sem_a[k]).wait()
            pltpu.make_async_copy(
                tab_b_h.at[idx_b[k]], rows_b[k], sem_b[k]).wait()

        def finish(c, k):
            drain(k)
            _dot_sigmoid_chunk(rows_a[k], rows_b[k], outbuf)
            pltpu.sync_copy(outbuf, out_h.at[pl.ds(base_w + c * CH, CH)])

        issue(0, 0)

        def pair(p, carry):
            c0 = p * 2
            issue(c0 + 1, 1)
            finish(c0, 0)
            @pl.when(c0 + 2 < nch)
            def _():
                issue(c0 + 2, 0)
            finish(c0 + 1, 1)
            return carry

        lax.fori_loop(0, nch // 2, pair, None)

    return run


def kernel(B, V, W, user_idx, item_idx, user_idx2, trust_idx2):
    pred_r = _make_pass(user_idx.shape[0])(B, V, user_idx, item_idx)
    pred_t = _make_pass(user_idx2.shape[0])(B, W, user_idx2, trust_idx2)
    return (pred_r, pred_t)


# speedup vs baseline: 1.1669x; 1.0653x over previous
import functools

import jax
import jax.numpy as jnp
from jax import lax
from jax.experimental import pallas as pl
from jax.experimental.pallas import tpu as pltpu
from jax.experimental.pallas import tpu_sc as plsc

D = 32
L = 16
NW = 32
CH = 640


def _dot_sigmoid_chunk(rows_a, rows_b, outbuf):
    def group(g, carry):
        row_ids = g * L + lax.iota(jnp.int32, L)
        lane = lax.iota(jnp.int32, L)
        accs = [jnp.zeros((L,), jnp.float32) for _ in range(4)]
        for d in range(D):
            col = (lane + d) & (D - 1)
            a = plsc.load_gather(rows_a, [row_ids, col])
            b = plsc.load_gather(rows_b, [row_ids, col])
            accs[d % 4] = accs[d % 4] + a * b
        acc = (accs[0] + accs[1]) + (accs[2] + accs[3])
        outbuf[pl.ds(g * L, L)] = 1.0 / (1.0 + jnp.exp(-acc))
        return carry

    lax.fori_loop(0, CH // L, group, None)


def _make_pass(n):
    assert n % (NW * 2 * CH) == 0
    mesh = plsc.VectorSubcoreMesh(core_axis_name="c", subcore_axis_name="s")

    @functools.partial(
        pl.kernel,
        out_type=jax.ShapeDtypeStruct((n,), jnp.float32),
        mesh=mesh,
        compiler_params=pltpu.CompilerParams(
            needs_layout_passes=False, use_tc_tiling_on_sc=False),
        scratch_types=[
            pltpu.VMEM((CH,), jnp.int32),
            pltpu.VMEM((CH,), jnp.int32),
            pltpu.VMEM((CH,), jnp.int32),
            pltpu.VMEM((CH,), jnp.int32),
            pltpu.VMEM((CH, D), jnp.float32),
            pltpu.VMEM((CH, D), jnp.float32),
            pltpu.VMEM((CH, D), jnp.float32),
            pltpu.VMEM((CH, D), jnp.float32),
            pltpu.VMEM((CH,), jnp.float32),
            pltpu.SemaphoreType.DMA,
            pltpu.SemaphoreType.DMA,
            pltpu.SemaphoreType.DMA,
            pltpu.SemaphoreType.DMA,
        ],
    )
    def run(tab_a_h, tab_b_h, ia_h, ib_h, out_h,
            ia0, ia1, ib0, ib1, ra0, ra1, rb0, rb1, outbuf,
            sa0, sa1, sb0, sb1):
        wid = lax.axis_index("s") * 2 + lax.axis_index("c")
        idx_a = (ia0, ia1)
        idx_b = (ib0, ib1)
        rows_a = (ra0, ra1)
        rows_b = (rb0, rb1)
        sem_a = (sa0, sa1)
        sem_b = (sb0, sb1)

        per_w = n // NW
        nch = per_w // CH
        base_w = wid * per_w

        def issue(c, k):
            base = base_w + c * CH
            pltpu.sync_copy(ia_h.at[pl.ds(base, CH)], idx_a[k])
            pltpu.sync_copy(ib_h.at[pl.ds(base, CH)], idx_b[k])
            pltpu.async_copy(tab_a_h.at[idx_a[k]], rows_a[k], sem_a[k])
            pltpu.async_copy(tab_b_h.at[idx_b[k]], rows_b[k], sem_b[k])

        def drain(k):
            pltpu.make_async_copy(
                tab_a_h.at[idx_a[k]], rows_a[k], sem_a[k]).wait()
            pltpu.make_async_copy(
                tab_b_h.at[idx_b[k]], rows_b[k], sem_b[k]).wait()

        def finish(c, k):
            drain(k)
            _dot_sigmoid_chunk(rows_a[k], rows_b[k], outbuf)
            pltpu.sync_copy(outbuf, out_h.at[pl.ds(base_w + c * CH, CH)])

        issue(0, 0)

        def pair(p, carry):
            c0 = p * 2
            issue(c0 + 1, 1)
            finish(c0, 0)
            @pl.when(c0 + 2 < nch)
            def _():
                issue(c0 + 2, 0)
            finish(c0 + 1, 1)
            return carry

        lax.fori_loop(0, nch // 2, pair, None)

    return run


def kernel(B, V, W, user_idx, item_idx, user_idx2, trust_idx2):
    pred_r = _make_pass(user_idx.shape[0])(B, V, user_idx, item_idx)
    z = jnp.minimum(pred_r[0], 0.0).astype(jnp.int32)
    pred_t = _make_pass(user_idx2.shape[0])(B, W, user_idx2 + z, trust_idx2)
    return (pred_r, pred_t)
